# baseline (speedup 1.0000x reference)
import jax
import jax.numpy as jnp
from jax.experimental import pallas as pl
from jax.experimental.pallas import tpu as pltpu

_R = 128
_B = 16384
_C = 256
_N = 32768


def _scan_body(x_ref, tri_ref, o_ref, carry_ref):
    i = pl.program_id(0)

    @pl.when(i == 0)
    def _():
        carry_ref[...] = jnp.zeros_like(carry_ref)

    tri = tri_ref[...]
    off = carry_ref[:, 0:1]
    for c in range(_B // _C):
        blk = x_ref[:, c * _C:(c + 1) * _C]
        hi = blk.astype(jnp.bfloat16)
        cs = jax.lax.dot(hi, tri, preferred_element_type=jnp.float32)
        o_ref[:, c * _C:(c + 1) * _C] = cs + off
        off = off + cs[:, _C - 1:_C]
    carry_ref[...] = jnp.broadcast_to(off, carry_ref.shape)


def kernel(x):
    tri = jnp.triu(jnp.ones((_C, _C), dtype=jnp.bfloat16))
    grid = (_N // _B,)
    return pl.pallas_call(
        _scan_body,
        grid=grid,
        in_specs=[
            pl.BlockSpec((_R, _B), lambda i: (0, i)),
            pl.BlockSpec((_C, _C), lambda i: (0, 0)),
        ],
        out_specs=pl.BlockSpec((_R, _B), lambda i: (0, i)),
        out_shape=jax.ShapeDtypeStruct((_R, _N), jnp.float32),
        scratch_shapes=[pltpu.VMEM((_R, 128), jnp.float32)],
        compiler_params=pltpu.CompilerParams(
            dimension_semantics=("arbitrary",),
        ),
    )(x, tri)


# speedup vs baseline: 1.4019x; 1.4019x over previous
import jax
import jax.numpy as jnp
from jax.experimental import pallas as pl
from jax.experimental.pallas import tpu as pltpu

_R = 128
_B = 16384
_C = 256
_N = 32768


def _scan_body(x_ref, tri_ref, o_ref, carry_ref):
    i = pl.program_id(0)

    @pl.when(i == 0)
    def _():
        carry_ref[...] = jnp.zeros_like(carry_ref)

    tri = tri_ref[...]
    off = carry_ref[:, 0:1]
    for c in range(_B // _C):
        blk = x_ref[:, c * _C:(c + 1) * _C]
        hi = blk.astype(jnp.bfloat16)
        lo = (blk - hi.astype(jnp.float32)).astype(jnp.bfloat16)
        cs = jax.lax.dot(hi, tri, preferred_element_type=jnp.float32)
        cs = cs + jax.lax.dot(lo, tri, preferred_element_type=jnp.float32)
        o_ref[:, c * _C:(c + 1) * _C] = cs + off
        off = off + cs[:, _C - 1:_C]
    carry_ref[...] = jnp.broadcast_to(off, carry_ref.shape)


def kernel(x):
    tri = jnp.triu(jnp.ones((_C, _C), dtype=jnp.bfloat16))
    grid = (_N // _B,)
    return pl.pallas_call(
        _scan_body,
        grid=grid,
        in_specs=[
            pl.BlockSpec((_R, _B), lambda i: (0, i)),
            pl.BlockSpec((_C, _C), lambda i: (0, 0)),
        ],
        out_specs=pl.BlockSpec((_R, _B), lambda i: (0, i)),
        out_shape=jax.ShapeDtypeStruct((_R, _N), jnp.float32),
        scratch_shapes=[pltpu.VMEM((_R, 128), jnp.float32)],
        compiler_params=pltpu.CompilerParams(
            dimension_semantics=("arbitrary",),
        ),
    )(x, tri)
